# baseline (speedup 1.0000x reference)
import functools

import jax
import jax.numpy as jnp
from jax import lax
from jax.experimental import pallas as pl
from jax.experimental.pallas import tpu as pltpu
from jax.experimental.pallas import tpu_sc as plsc

B = 16384
F = 32
NC = 2
NS = 16
NW = NC * NS
BPW = B // NW
CHUNK = 128
NCHUNK = BPW // CHUNK
L = 16


def _sc_body(users_hbm, movies_hbm, uemb_hbm, memb_hbm, ubias_hbm, mbias_hbm,
             out_hbm, uidx_v, midx_v, urows_v, mrows_v, ubias_v, mbias_v,
             dot_v, sem):
    wid = lax.axis_index("s") * NC + lax.axis_index("c")
    base = wid * BPW

    for j in range(NCHUNK):
        off = base + j * CHUNK
        pltpu.sync_copy(users_hbm.at[pl.ds(off, CHUNK)], uidx_v.at[j])
        pltpu.sync_copy(movies_hbm.at[pl.ds(off, CHUNK)], midx_v.at[j])

    copies = []
    for j in range(NCHUNK):
        dst = pl.ds(j * CHUNK, CHUNK)
        copies.append(pltpu.async_copy(uemb_hbm.at[uidx_v.at[j]],
                                       urows_v.at[dst], sem))
        copies.append(pltpu.async_copy(memb_hbm.at[midx_v.at[j]],
                                       mrows_v.at[dst], sem))
        copies.append(pltpu.async_copy(ubias_hbm.at[uidx_v.at[j]],
                                       ubias_v.at[dst], sem))
        copies.append(pltpu.async_copy(mbias_hbm.at[midx_v.at[j]],
                                       mbias_v.at[dst], sem))
    for c in copies:
        c.wait()

    lane = lax.iota(jnp.int32, L)

    def group(g, carry):
        rows = g * L + lane
        s = pl.ds(g * L, L)
        acc = ubias_v[s] + mbias_v[s]
        for f in range(F):
            fv = jnp.full((L,), f, jnp.int32)
            acc += (plsc.load_gather(urows_v, [rows, fv]) *
                    plsc.load_gather(mrows_v, [rows, fv]))
        dot_v[s] = 1.0 / (1.0 + jnp.exp(-acc))
        return carry

    lax.fori_loop(0, BPW // L, group, 0, unroll=2)

    pltpu.sync_copy(dot_v, out_hbm.at[pl.ds(base, BPW)])


@jax.jit
def _mf_sc(users, movies, uemb, memb, ubias1d, mbias1d):
    mesh = plsc.VectorSubcoreMesh(core_axis_name="c", subcore_axis_name="s")
    return pl.kernel(
        _sc_body,
        out_type=jax.ShapeDtypeStruct((B,), jnp.float32),
        mesh=mesh,
        compiler_params=pltpu.CompilerParams(needs_layout_passes=False,
                                             use_tc_tiling_on_sc=False),
        scratch_types=[
            pltpu.VMEM((NCHUNK, CHUNK), jnp.int32),
            pltpu.VMEM((NCHUNK, CHUNK), jnp.int32),
            pltpu.VMEM((BPW, F), jnp.float32),
            pltpu.VMEM((BPW, F), jnp.float32),
            pltpu.VMEM((BPW,), jnp.float32),
            pltpu.VMEM((BPW,), jnp.float32),
            pltpu.VMEM((BPW,), jnp.float32),
            pltpu.SemaphoreType.DMA,
        ],
    )(users, movies, uemb, memb, ubias1d, mbias1d)


def kernel(users, movies, user_embedding, movie_embedding, user_bias,
           movie_bias):
    return _mf_sc(users, movies, user_embedding, movie_embedding,
                  user_bias.reshape(-1), movie_bias.reshape(-1))
